# Initial kernel scaffold; baseline (speedup 1.0000x reference)
import jax
import jax.numpy as jnp
from jax.experimental import pallas as pl


def kernel(x, embedding, ema_w, ema_cluster_size, running_mean, running_var):
    raise NotImplementedError("write your pallas kernel here")


# speedup vs baseline: 1.1364x; 1.1364x over previous
import functools

import jax
import jax.numpy as jnp
from jax.experimental import pallas as pl
from jax.experimental.pallas import tpu as pltpu

_N, _K, _D = 8192, 8192, 256
_BN = 1024
_BK = 2048


def _mean_kernel(x_ref, mean_ref):
    mean_ref[...] = jnp.sum(x_ref[...], axis=0, keepdims=True) * (1.0 / _N)


def _var_kernel(x_ref, mean_ref, var_ref):
    d = x_ref[...] - mean_ref[...]
    var_ref[...] = jnp.sum(d * d, axis=0, keepdims=True) * (1.0 / _N)


def _argmin_kernel(x_ref, e_ref, mean_ref, var_ref, out_ref,
                   xnb_ref, rn_ref, bv_ref, bi_ref):
    i = pl.program_id(0)
    j = pl.program_id(1)

    @pl.when(j == 0)
    def _():
        xb = x_ref[...]
        xn = (xb - mean_ref[...]) / jnp.sqrt(var_ref[...] + 1e-5)
        xnb_ref[...] = (xn * (-2.0)).astype(jnp.bfloat16)
        rn_ref[...] = jnp.sum(xn * xn, axis=1, keepdims=True)

    e = e_ref[...]
    esq = jnp.sum(e * e, axis=1)
    dot = jax.lax.dot_general(
        xnb_ref[...], e.astype(jnp.bfloat16), (((1,), (1,)), ((), ())),
        preferred_element_type=jnp.float32)
    dist = (rn_ref[...] + esq[None, :]) + dot
    lmin = jnp.min(dist, axis=1, keepdims=True)
    ids = jax.lax.broadcasted_iota(jnp.int32, dist.shape, 1)
    larg = jnp.min(jnp.where(dist == lmin, ids, _BK), axis=1,
                   keepdims=True) + j * _BK

    @pl.when(j == 0)
    def _():
        bv_ref[...] = lmin.astype(jnp.bfloat16).astype(jnp.float32)
        bi_ref[...] = larg

    @pl.when(j > 0)
    def _():
        upd = lmin < bv_ref[...]
        nv = jnp.where(upd, lmin, bv_ref[...])
        bv_ref[...] = nv.astype(jnp.bfloat16).astype(jnp.float32)
        bi_ref[...] = jnp.where(upd, larg, bi_ref[...])

    @pl.when(j == pl.num_programs(1) - 1)
    def _():
        out_ref[pl.ds(i * _BN, _BN), :] = bi_ref[...]


@jax.jit
def _encode(x, embedding):
    mean = pl.pallas_call(
        _mean_kernel,
        grid=(1,),
        in_specs=[pl.BlockSpec((_N, _D), lambda i: (0, 0))],
        out_specs=pl.BlockSpec((1, _D), lambda i: (0, 0)),
        out_shape=jax.ShapeDtypeStruct((1, _D), jnp.float32),
    )(x)
    var = pl.pallas_call(
        _var_kernel,
        grid=(1,),
        in_specs=[pl.BlockSpec((_N, _D), lambda i: (0, 0)),
                  pl.BlockSpec((1, _D), lambda i: (0, 0))],
        out_specs=pl.BlockSpec((1, _D), lambda i: (0, 0)),
        out_shape=jax.ShapeDtypeStruct((1, _D), jnp.float32),
    )(x, mean)

    idx = pl.pallas_call(
        _argmin_kernel,
        grid=(_N // _BN, _K // _BK),
        in_specs=[pl.BlockSpec((_BN, _D), lambda i, j: (i, 0)),
                  pl.BlockSpec((_BK, _D), lambda i, j: (j, 0)),
                  pl.BlockSpec((1, _D), lambda i, j: (0, 0)),
                  pl.BlockSpec((1, _D), lambda i, j: (0, 0))],
        out_specs=pl.BlockSpec((_N, 1), lambda i, j: (0, 0)),
        out_shape=jax.ShapeDtypeStruct((_N, 1), jnp.int32),
        scratch_shapes=[pltpu.VMEM((_BN, _D), jnp.bfloat16),
                        pltpu.VMEM((_BN, 1), jnp.float32),
                        pltpu.VMEM((_BN, 1), jnp.float32),
                        pltpu.VMEM((_BN, 1), jnp.int32)],
    )(x, embedding, mean, var)
    return idx


def kernel(x, embedding, ema_w, ema_cluster_size, running_mean, running_var):
    return _encode(x, embedding)
